# Initial kernel scaffold; baseline (speedup 1.0000x reference)
import functools

import jax
import jax.numpy as jnp
from jax import lax
from jax.experimental import pallas as pl
from jax.experimental.pallas import tpu as pltpu
from jax.experimental.pallas import tpu_sc as plsc

N = 50000
E = 800000
D_IN = 128
H = 64
C = 4
B = 64

HALF = 25088
NP = 2 * HALF
TRASH = 25080
BLK = 512
NBLK = NP // BLK

NTILE = 16
NCORE = 2
EPT = 50048
EPAD = NTILE * EPT
KCH = 128
NCHUNK = EPT // KCH
EPW = EPAD // (NTILE * NCORE)
ROWS_PT = HALF // NTILE

_mesh = plsc.VectorSubcoreMesh(core_axis_name="c", subcore_axis_name="s")


@functools.partial(
    pl.kernel,
    out_type=jax.ShapeDtypeStruct((NTILE * NCORE, NP), jnp.float32),
    mesh=_mesh,
    scratch_types=[
        pltpu.VMEM((NP,), jnp.float32),
        pltpu.VMEM((EPW,), jnp.int32),
    ],
)
def _deg_kernel(dst_hbm, out_hbm, hist, dbuf):
    wid = lax.axis_index("s") * NCORE + lax.axis_index("c")

    def zero_body(i, _):
        hist[pl.ds(i * 16, 16)] = jnp.zeros((16,), jnp.float32)
        return 0

    lax.fori_loop(0, NP // 16, zero_body, 0)
    pltpu.sync_copy(dst_hbm.at[pl.ds(wid * EPW, EPW)], dbuf)
    ones = jnp.ones((16,), jnp.float32)

    def body(i, _):
        d = dbuf[pl.ds(i * 16, 16)]
        ok = d >= 0
        idx = jnp.where(d >= N // 2, d + (HALF - N // 2), d)
        idx = jnp.where(ok, idx, 0)
        plsc.addupdate_scatter(hist, [idx], ones, mask=ok)
        return 0

    lax.fori_loop(0, EPW // 16, body, 0)
    pltpu.sync_copy(hist, out_hbm.at[wid])


@functools.partial(
    pl.kernel,
    out_type=jax.ShapeDtypeStruct((NCORE, HALF, H), jnp.float32),
    mesh=_mesh,
    scratch_types=[
        pltpu.VMEM_SHARED((HALF, H), jnp.float32),
        pltpu.VMEM((KCH, H), jnp.float32),
        pltpu.VMEM((KCH, H), jnp.float32),
        pltpu.VMEM((KCH,), jnp.int32),
        pltpu.VMEM((KCH,), jnp.int32),
        pltpu.VMEM((KCH,), jnp.int32),
        pltpu.VMEM((KCH,), jnp.int32),
        pltpu.SemaphoreType.DMA,
    ],
)
def _seg_kernel(src_hbm, dst_hbm, xs_hbm, out_hbm,
                acc, rows, zbuf, sbuf, dbuf, gidx, sidx, sem):
    c = lax.axis_index("c")
    s = lax.axis_index("s")

    def zrow(r, _):
        for jj in range(H // 16):
            zbuf[r, pl.ds(jj * 16, 16)] = jnp.zeros((16,), jnp.float32)
        return 0

    lax.fori_loop(0, KCH, zrow, 0)
    base = s * ROWS_PT
    for k in range(ROWS_PT // KCH):
        pltpu.sync_copy(zbuf, acc.at[pl.ds(base + k * KCH, KCH)])
    rem = ROWS_PT % KCH
    if rem:
        pltpu.sync_copy(zbuf.at[pl.ds(0, rem)],
                        acc.at[pl.ds(base + (ROWS_PT // KCH) * KCH, rem)])
    plsc.subcore_barrier()

    ebase = s * EPT
    half_lo = c * (N // 2)

    def chunk(i, _):
        off = ebase + i * KCH
        pltpu.sync_copy(src_hbm.at[pl.ds(off, KCH)], sbuf)
        pltpu.sync_copy(dst_hbm.at[pl.ds(off, KCH)], dbuf)
        for j in range(KCH // 16):
            sl = pl.ds(j * 16, 16)
            sv = sbuf[sl]
            gidx[sl] = jnp.where(sv >= N // 2, sv + (HALF - N // 2), sv)
            loc = dbuf[sl] - half_lo
            own = (loc >= 0) & (loc < N // 2)
            sidx[sl] = jnp.where(own, loc, TRASH)
        pltpu.async_copy(xs_hbm.at[gidx], rows, sem).wait()
        pltpu.sync_copy(rows, acc.at[sidx], add=True)
        return 0

    lax.fori_loop(0, NCHUNK, chunk, 0)
    plsc.subcore_barrier()
    pltpu.sync_copy(acc.at[pl.ds(base, ROWS_PT)],
                    out_hbm.at[c, pl.ds(base, ROWS_PT)])


def _tcb_body(x_ref, hist_ref, w1_ref, xs_ref, dinv_ref):
    deg = jnp.sum(hist_ref[...], axis=0) + 1.0
    dinv = lax.rsqrt(deg)
    xw = jnp.dot(x_ref[...], w1_ref[...], preferred_element_type=jnp.float32)
    xs_ref[...] = xw * dinv[:, None]
    dinv_ref[...] = dinv[:, None]


def _tcb(x_pad, hist, W1):
    return pl.pallas_call(
        _tcb_body,
        grid=(NBLK,),
        in_specs=[
            pl.BlockSpec((BLK, D_IN), lambda i: (i, 0)),
            pl.BlockSpec((NTILE * NCORE, BLK), lambda i: (0, i)),
            pl.BlockSpec((D_IN, H), lambda i: (0, 0)),
        ],
        out_specs=[
            pl.BlockSpec((BLK, H), lambda i: (i, 0)),
            pl.BlockSpec((BLK, 1), lambda i: (i, 0)),
        ],
        out_shape=[
            jax.ShapeDtypeStruct((NP, H), jnp.float32),
            jax.ShapeDtypeStruct((NP, 1), jnp.float32),
        ],
    )(x_pad, hist, W1)


def _tcc_body(seg_ref, xs_ref, dinv_ref, b_ref, w2_ref, out_ref):
    dinv = dinv_ref[...]
    h = jax.nn.relu(dinv * (seg_ref[...] + xs_ref[...]) + b_ref[...])
    xw = jnp.dot(h, w2_ref[...], preferred_element_type=jnp.float32)
    out_ref[...] = xw * dinv


def _tcc(seg1, xs1, dinv, b1, W2):
    return pl.pallas_call(
        _tcc_body,
        grid=(NBLK,),
        in_specs=[
            pl.BlockSpec((BLK, H), lambda i: (i, 0)),
            pl.BlockSpec((BLK, H), lambda i: (i, 0)),
            pl.BlockSpec((BLK, 1), lambda i: (i, 0)),
            pl.BlockSpec((1, H), lambda i: (0, 0)),
            pl.BlockSpec((H, H), lambda i: (0, 0)),
        ],
        out_specs=pl.BlockSpec((BLK, H), lambda i: (i, 0)),
        out_shape=jax.ShapeDtypeStruct((NP, H), jnp.float32),
    )(seg1, xs1, dinv, b1, W2)


def _tcd_body(seg_ref, xs_ref, dinv_ref, b_ref, batch_ref,
              wf1_ref, bf1_ref, wf2_ref, bf2_ref, out_ref, sums, cnts):
    i = pl.program_id(0)

    @pl.when(i == 0)
    def _():
        sums[...] = jnp.zeros_like(sums)
        cnts[...] = jnp.zeros_like(cnts)

    h = jax.nn.relu(dinv_ref[...] * (seg_ref[...] + xs_ref[...]) + b_ref[...])
    bv = batch_ref[0, 0, :]
    m = (bv[:, None] == lax.broadcasted_iota(jnp.int32, (BLK, B), 1))
    mf = m.astype(jnp.float32)
    dn = (((0,), (0,)), ((), ()))
    sums[...] += lax.dot_general(mf, h, dn, preferred_element_type=jnp.float32)
    cnts[...] += lax.dot_general(mf, jnp.ones((BLK, 8), jnp.float32), dn,
                                 preferred_element_type=jnp.float32)

    @pl.when(i == NBLK - 1)
    def _():
        g = sums[...] / jnp.maximum(cnts[...][:, :1], 1.0)
        a = jax.nn.relu(
            jnp.dot(g, wf1_ref[...], preferred_element_type=jnp.float32)
            + bf1_ref[...])
        out_ref[...] = (
            jnp.dot(a, wf2_ref[...], preferred_element_type=jnp.float32)
            + bf2_ref[...])


def _tcd(seg2, xs2, dinv, b2, batch3, Wf1, bf1, Wf2, bf2):
    return pl.pallas_call(
        _tcd_body,
        grid=(NBLK,),
        in_specs=[
            pl.BlockSpec((BLK, H), lambda i: (i, 0)),
            pl.BlockSpec((BLK, H), lambda i: (i, 0)),
            pl.BlockSpec((BLK, 1), lambda i: (i, 0)),
            pl.BlockSpec((1, H), lambda i: (0, 0)),
            pl.BlockSpec((1, 1, BLK), lambda i: (i, 0, 0)),
            pl.BlockSpec((H, H), lambda i: (0, 0)),
            pl.BlockSpec((1, H), lambda i: (0, 0)),
            pl.BlockSpec((H, C), lambda i: (0, 0)),
            pl.BlockSpec((1, C), lambda i: (0, 0)),
        ],
        out_specs=pl.BlockSpec((B, C), lambda i: (0, 0)),
        out_shape=jax.ShapeDtypeStruct((B, C), jnp.float32),
        scratch_shapes=[
            pltpu.VMEM((B, B), jnp.float32),
            pltpu.VMEM((B, 8), jnp.float32),
        ],
    )(seg2, xs2, dinv, b2, batch3, Wf1, bf1, Wf2, bf2)


@jax.jit
def kernel(x, edge_index, batch, W1, b1, W2, b2, Wf1, bf1, Wf2, bf2):
    src = edge_index[0].astype(jnp.int32)
    dst = edge_index[1].astype(jnp.int32)
    pad_e = EPAD - E
    src_pad = jnp.concatenate([src, jnp.zeros((pad_e,), jnp.int32)])
    dst_pad = jnp.concatenate([dst, jnp.full((pad_e,), -1, jnp.int32)])

    half = N // 2
    gap = HALF - half
    x_pad = jnp.concatenate([
        x[:half], jnp.zeros((gap, D_IN), jnp.float32),
        x[half:], jnp.zeros((gap, D_IN), jnp.float32)])
    b32 = batch.astype(jnp.int32)
    batch_pad = jnp.concatenate([
        b32[:half], jnp.full((gap,), B, jnp.int32),
        b32[half:], jnp.full((gap,), B, jnp.int32)])
    batch3 = batch_pad.reshape(NBLK, 1, BLK)

    b1r = b1.reshape(1, H)
    b2r = b2.reshape(1, H)
    bf1r = bf1.reshape(1, H)
    bf2r = bf2.reshape(1, C)

    hist = _deg_kernel(dst_pad)
    xs1, dinv = _tcb(x_pad, hist, W1)
    seg1 = _seg_kernel(src_pad, dst_pad, xs1).reshape(NP, H)
    xs2 = _tcc(seg1, xs1, dinv, b1r, W2)
    seg2 = _seg_kernel(src_pad, dst_pad, xs2).reshape(NP, H)
    return _tcd(seg2, xs2, dinv, b2r, batch3, Wf1, bf1r, Wf2, bf2r)


# speedup vs baseline: 8.5023x; 8.5023x over previous
import functools

import jax
import jax.numpy as jnp
from jax import lax
from jax.experimental import pallas as pl
from jax.experimental.pallas import tpu as pltpu
from jax.experimental.pallas import tpu_sc as plsc

N = 50000
E = 800000
D_IN = 128
H = 64
C = 4
B = 64

HALF = 25088
NP = 2 * HALF
NQ = 8
QR = NP // NQ
TRASH_Q = QR
ACC_R = 6400
BLK = 512
NBLK = NP // BLK

NTILE = 16
NCORE = 2
NW = NTILE * NCORE
EPAD = 800768
EPW = EPAD // NW
KCH = 128
R_CAP = 4096
PEND = 272
ZPT = ACC_R // NTILE
WPT = QR // NTILE

_mesh = plsc.VectorSubcoreMesh(core_axis_name="c", subcore_axis_name="s")


@functools.partial(
    pl.kernel,
    out_type=[
        jax.ShapeDtypeStruct((NQ * NW * R_CAP,), jnp.int32),
        jax.ShapeDtypeStruct((NQ * NW * R_CAP,), jnp.int32),
        jax.ShapeDtypeStruct((NW * 128,), jnp.int32),
    ],
    mesh=_mesh,
    scratch_types=[
        pltpu.VMEM((EPW,), jnp.int32),
        pltpu.VMEM((EPW,), jnp.int32),
    ] + [pltpu.VMEM((PEND,), jnp.int32)] * 16 + [
        pltpu.VMEM((KCH,), jnp.int32),
        pltpu.SMEM((16,), jnp.int32),
    ],
    compiler_params=pltpu.CompilerParams(needs_layout_passes=False),
)
def _part_kernel(src_hbm, dst_hbm, gsrc_hbm, ldst_hbm, counts_hbm,
                 sbuf, dbuf, *rest):
    pends = rest[:16]
    cbuf = rest[16]
    cnts = rest[17]
    psrcs = pends[:NQ]
    pdsts = pends[NQ:]
    wid = lax.axis_index("s") * NCORE + lax.axis_index("c")
    pltpu.sync_copy(src_hbm.at[pl.ds(pl.multiple_of(wid * EPW, 128), EPW)], sbuf)
    pltpu.sync_copy(dst_hbm.at[pl.ds(pl.multiple_of(wid * EPW, 128), EPW)], dbuf)
    for b in range(16):
        cnts[b] = 0

    def vec(i, _):
        sl = pl.ds(i * 16, 16)
        sv = sbuf[sl]
        dv = dbuf[sl]
        g = jnp.where(sv >= N // 2, sv + (HALF - N // 2), sv)
        gd = jnp.where(dv >= N // 2, dv + (HALF - N // 2), dv)
        q = (gd >= QR).astype(jnp.int32)
        for k in range(2, NQ):
            q = q + (gd >= k * QR).astype(jnp.int32)
        valid = dv >= 0
        for b in range(NQ):
            m = (q == b) & valid
            mi = m.astype(jnp.int32)
            psrc = psrcs[b]
            pdst = pdsts[b]
            cnt = cnts[b]
            rank = cnt + plsc.cumsum(mi) - 1
            plsc.store_scatter(psrc, [rank], g, mask=m)
            plsc.store_scatter(pdst, [rank], gd - b * QR, mask=m)
            cnt = cnt + jnp.sum(mi)
            cnts[b] = cnt

            @pl.when(cnt >= KCH)
            def _():
                out = pl.multiple_of(
                    jnp.minimum(cnts[NQ + b], R_CAP - KCH), KCH)
                rbase = pl.multiple_of((b * NW + wid) * R_CAP, KCH)
                pltpu.sync_copy(psrc.at[pl.ds(0, KCH)],
                                gsrc_hbm.at[pl.ds(rbase + out, KCH)])
                pltpu.sync_copy(pdst.at[pl.ds(0, KCH)],
                                ldst_hbm.at[pl.ds(rbase + out, KCH)])
                psrc[pl.ds(0, 16)] = psrc[pl.ds(KCH, 16)]
                pdst[pl.ds(0, 16)] = pdst[pl.ds(KCH, 16)]
                cnts[b] = cnt - KCH
                cnts[NQ + b] = out + KCH

        return 0

    lax.fori_loop(0, EPW // 16, vec, 0)

    zsrc = jnp.zeros((16,), jnp.int32)
    ztrash = jnp.full((16,), TRASH_Q, jnp.int32)
    lanes = lax.iota(jnp.int32, 16)
    cvec = jnp.zeros((16,), jnp.int32)
    for b in range(NQ):
        psrc = psrcs[b]
        pdst = pdsts[b]
        cnt = cnts[b]
        for i in range(8):
            fidx = cnt + i * 16 + lanes
            plsc.store_scatter(psrc, [fidx], zsrc)
            plsc.store_scatter(pdst, [fidx], ztrash)
        out = pl.multiple_of(jnp.minimum(cnts[NQ + b], R_CAP - KCH), KCH)
        rbase = pl.multiple_of((b * NW + wid) * R_CAP, KCH)
        pltpu.sync_copy(psrc.at[pl.ds(0, KCH)],
                        gsrc_hbm.at[pl.ds(rbase + out, KCH)])
        pltpu.sync_copy(pdst.at[pl.ds(0, KCH)],
                        ldst_hbm.at[pl.ds(rbase + out, KCH)])
        cvec = jnp.where(lanes == b, jnp.minimum(cnts[NQ + b], R_CAP - KCH) + KCH, cvec)
    for i in range(KCH // 16):
        cbuf[pl.ds(i * 16, 16)] = cvec
    pltpu.sync_copy(cbuf, counts_hbm.at[pl.ds(pl.multiple_of(wid * 128, 128), KCH)])


def _zero_fill(buf, nrow, val=0.0):
    def row(r, _):
        for jj in range(8):
            buf[r, pl.ds(jj * 16, 16)] = jnp.full((16,), val, jnp.float32)
        return 0

    lax.fori_loop(0, nrow, row, 0)


def _zero_acc(acc, zbuf, s):
    base = s * ZPT
    for k in range(ZPT // KCH):
        pltpu.sync_copy(zbuf, acc.at[pl.ds(base + k * KCH, KCH)])
    rem = ZPT % KCH
    if rem:
        pltpu.sync_copy(zbuf.at[pl.ds(0, rem)],
                        acc.at[pl.ds(base + (ZPT // KCH) * KCH, rem)])


@functools.partial(
    pl.kernel,
    out_type=jax.ShapeDtypeStruct((NP, 2 * H), jnp.float32),
    mesh=_mesh,
    scratch_types=[
        pltpu.VMEM_SHARED((ACC_R, 2 * H), jnp.float32),
        pltpu.VMEM((KCH, 2 * H), jnp.float32),
        pltpu.VMEM((KCH, 2 * H), jnp.float32),
        pltpu.VMEM((NW * 128,), jnp.int32),
        pltpu.VMEM((KCH,), jnp.int32),
    ],
    compiler_params=pltpu.CompilerParams(needs_layout_passes=False),
)
def _deg_kernel(ldst_hbm, counts_hbm, out_hbm, acc, obuf, zbuf, cvm, sidx):
    c = lax.axis_index("c")
    s = lax.axis_index("s")
    pltpu.sync_copy(counts_hbm, cvm)
    _zero_fill(obuf, KCH, 1.0)
    _zero_fill(zbuf, KCH, 0.0)

    for qq in range(NQ // 2):
        q = c * (NQ // 2) + qq
        _zero_acc(acc, zbuf, s)
        plsc.subcore_barrier()
        for rr in range(2):
            reg = s * 2 + rr
            crow = cvm[pl.ds(pl.multiple_of(reg * 128, 128), 16)]
            lanes = lax.iota(jnp.int32, 16)
            nch = jnp.sum(jnp.where(lanes == q, crow, 0)) // KCH
            nch = jnp.minimum(nch, R_CAP // KCH)
            rbase = pl.multiple_of((q * NW + reg) * R_CAP, KCH)

            def chunk(i, _):
                off = pl.multiple_of(rbase + i * KCH, KCH)
                pltpu.sync_copy(ldst_hbm.at[pl.ds(off, KCH)], sidx)
                pltpu.sync_copy(obuf, acc.at[sidx], add=True)
                return 0

            lax.fori_loop(0, nch, chunk, 0)
        plsc.subcore_barrier()
        pltpu.sync_copy(acc.at[pl.ds(s * WPT, WPT)],
                        out_hbm.at[pl.ds(q * QR + s * WPT, WPT)])
        plsc.subcore_barrier()


@functools.partial(
    pl.kernel,
    out_type=jax.ShapeDtypeStruct((NP, 2 * H), jnp.float32),
    mesh=_mesh,
    scratch_types=[
        pltpu.VMEM_SHARED((ACC_R, 2 * H), jnp.float32),
        pltpu.VMEM((KCH, 2 * H), jnp.float32),
        pltpu.VMEM((KCH, 2 * H), jnp.float32),
        pltpu.VMEM((NW * 128,), jnp.int32),
        pltpu.VMEM((KCH,), jnp.int32),
        pltpu.VMEM((KCH,), jnp.int32),
        pltpu.SemaphoreType.DMA,
    ],
    compiler_params=pltpu.CompilerParams(needs_layout_passes=False),
)
def _seg_kernel(gsrc_hbm, ldst_hbm, counts_hbm, xs_hbm, out_hbm,
                acc, rows, zbuf, cvm, gidx, sidx, sem):
    c = lax.axis_index("c")
    s = lax.axis_index("s")
    pltpu.sync_copy(counts_hbm, cvm)
    _zero_fill(zbuf, KCH, 0.0)

    for qq in range(NQ // 2):
        q = c * (NQ // 2) + qq
        _zero_acc(acc, zbuf, s)
        plsc.subcore_barrier()
        for rr in range(2):
            reg = s * 2 + rr
            crow = cvm[pl.ds(pl.multiple_of(reg * 128, 128), 16)]
            lanes = lax.iota(jnp.int32, 16)
            nch = jnp.sum(jnp.where(lanes == q, crow, 0)) // KCH
            nch = jnp.minimum(nch, R_CAP // KCH)
            rbase = pl.multiple_of((q * NW + reg) * R_CAP, KCH)

            def chunk(i, _):
                off = pl.multiple_of(rbase + i * KCH, KCH)
                pltpu.sync_copy(gsrc_hbm.at[pl.ds(off, KCH)], gidx)
                pltpu.sync_copy(ldst_hbm.at[pl.ds(off, KCH)], sidx)
                pltpu.async_copy(xs_hbm.at[gidx], rows, sem).wait()
                pltpu.sync_copy(rows, acc.at[sidx], add=True)
                return 0

            lax.fori_loop(0, nch, chunk, 0)
        plsc.subcore_barrier()
        pltpu.sync_copy(acc.at[pl.ds(s * WPT, WPT)],
                        out_hbm.at[pl.ds(q * QR + s * WPT, WPT)])
        plsc.subcore_barrier()


def _tcb_body(x_ref, hist_ref, w1_ref, xs_ref, dinv_ref):
    deg = hist_ref[:, :1] + 1.0
    dinv = lax.rsqrt(deg)
    xw = jnp.dot(x_ref[...], w1_ref[...], preferred_element_type=jnp.float32)
    xs_ref[...] = jnp.concatenate(
        [xw * dinv, jnp.zeros((BLK, H), jnp.float32)], axis=1)
    dinv_ref[...] = dinv


def _tcb(x_pad, hist, W1):
    return pl.pallas_call(
        _tcb_body,
        grid=(NBLK,),
        in_specs=[
            pl.BlockSpec((BLK, D_IN), lambda i: (i, 0)),
            pl.BlockSpec((BLK, 2 * H), lambda i: (i, 0)),
            pl.BlockSpec((D_IN, H), lambda i: (0, 0)),
        ],
        out_specs=[
            pl.BlockSpec((BLK, 2 * H), lambda i: (i, 0)),
            pl.BlockSpec((BLK, 1), lambda i: (i, 0)),
        ],
        out_shape=[
            jax.ShapeDtypeStruct((NP, 2 * H), jnp.float32),
            jax.ShapeDtypeStruct((NP, 1), jnp.float32),
        ],
    )(x_pad, hist, W1)


def _tcc_body(seg_ref, xs_ref, dinv_ref, b_ref, w2_ref, out_ref):
    dinv = dinv_ref[...]
    h = jax.nn.relu(dinv * (seg_ref[:, :H] + xs_ref[:, :H]) + b_ref[...])
    xw = jnp.dot(h, w2_ref[...], preferred_element_type=jnp.float32)
    out_ref[...] = jnp.concatenate(
        [xw * dinv, jnp.zeros((BLK, H), jnp.float32)], axis=1)


def _tcc(seg1, xs1, dinv, b1, W2):
    return pl.pallas_call(
        _tcc_body,
        grid=(NBLK,),
        in_specs=[
            pl.BlockSpec((BLK, 2 * H), lambda i: (i, 0)),
            pl.BlockSpec((BLK, 2 * H), lambda i: (i, 0)),
            pl.BlockSpec((BLK, 1), lambda i: (i, 0)),
            pl.BlockSpec((1, H), lambda i: (0, 0)),
            pl.BlockSpec((H, H), lambda i: (0, 0)),
        ],
        out_specs=pl.BlockSpec((BLK, 2 * H), lambda i: (i, 0)),
        out_shape=jax.ShapeDtypeStruct((NP, 2 * H), jnp.float32),
    )(seg1, xs1, dinv, b1, W2)


def _tcd_body(seg_ref, xs_ref, dinv_ref, b_ref, batch_ref,
              wf1_ref, bf1_ref, wf2_ref, bf2_ref, out_ref, sums, cnts):
    i = pl.program_id(0)

    @pl.when(i == 0)
    def _():
        sums[...] = jnp.zeros_like(sums)
        cnts[...] = jnp.zeros_like(cnts)

    h = jax.nn.relu(dinv_ref[...] * (seg_ref[:, :H] + xs_ref[:, :H])
                    + b_ref[...])
    bv = batch_ref[0, 0, :]
    m = (bv[:, None] == lax.broadcasted_iota(jnp.int32, (BLK, B), 1))
    mf = m.astype(jnp.float32)
    dn = (((0,), (0,)), ((), ()))
    sums[...] += lax.dot_general(mf, h, dn, preferred_element_type=jnp.float32)
    cnts[...] += lax.dot_general(mf, jnp.ones((BLK, 8), jnp.float32), dn,
                                 preferred_element_type=jnp.float32)

    @pl.when(i == NBLK - 1)
    def _():
        g = sums[...] / jnp.maximum(cnts[...][:, :1], 1.0)
        a = jax.nn.relu(
            jnp.dot(g, wf1_ref[...], preferred_element_type=jnp.float32)
            + bf1_ref[...])
        out_ref[...] = (
            jnp.dot(a, wf2_ref[...], preferred_element_type=jnp.float32)
            + bf2_ref[...])


def _tcd(seg2, xs2, dinv, b2, batch3, Wf1, bf1, Wf2, bf2):
    return pl.pallas_call(
        _tcd_body,
        grid=(NBLK,),
        in_specs=[
            pl.BlockSpec((BLK, 2 * H), lambda i: (i, 0)),
            pl.BlockSpec((BLK, 2 * H), lambda i: (i, 0)),
            pl.BlockSpec((BLK, 1), lambda i: (i, 0)),
            pl.BlockSpec((1, H), lambda i: (0, 0)),
            pl.BlockSpec((1, 1, BLK), lambda i: (i, 0, 0)),
            pl.BlockSpec((H, H), lambda i: (0, 0)),
            pl.BlockSpec((1, H), lambda i: (0, 0)),
            pl.BlockSpec((H, C), lambda i: (0, 0)),
            pl.BlockSpec((1, C), lambda i: (0, 0)),
        ],
        out_specs=pl.BlockSpec((B, C), lambda i: (0, 0)),
        out_shape=jax.ShapeDtypeStruct((B, C), jnp.float32),
        scratch_shapes=[
            pltpu.VMEM((B, B), jnp.float32),
            pltpu.VMEM((B, 8), jnp.float32),
        ],
    )(seg2, xs2, dinv, b2, batch3, Wf1, bf1, Wf2, bf2)


@jax.jit
def kernel(x, edge_index, batch, W1, b1, W2, b2, Wf1, bf1, Wf2, bf2):
    src = edge_index[0].astype(jnp.int32)
    dst = edge_index[1].astype(jnp.int32)
    pad_e = EPAD - E
    src_pad = jnp.concatenate([src, jnp.zeros((pad_e,), jnp.int32)])
    dst_pad = jnp.concatenate([dst, jnp.full((pad_e,), -1, jnp.int32)])

    half = N // 2
    gap = HALF - half
    x_pad = jnp.concatenate([
        x[:half], jnp.zeros((gap, D_IN), jnp.float32),
        x[half:], jnp.zeros((gap, D_IN), jnp.float32)])
    b32 = batch.astype(jnp.int32)
    batch_pad = jnp.concatenate([
        b32[:half], jnp.full((gap,), B, jnp.int32),
        b32[half:], jnp.full((gap,), B, jnp.int32)])
    batch3 = batch_pad.reshape(NBLK, 1, BLK)

    b1r = b1.reshape(1, H)
    b2r = b2.reshape(1, H)
    bf1r = bf1.reshape(1, H)
    bf2r = bf2.reshape(1, C)

    gsrc, ldst, counts = _part_kernel(src_pad, dst_pad)
    hist = _deg_kernel(ldst, counts)
    xs1, dinv = _tcb(x_pad, hist, W1)
    seg1 = _seg_kernel(gsrc, ldst, counts, xs1)
    xs2 = _tcc(seg1, xs1, dinv, b1r, W2)
    seg2 = _seg_kernel(gsrc, ldst, counts, xs2)
    return _tcd(seg2, xs2, dinv, b2r, batch3, Wf1, bf1r, Wf2, bf2r)


# speedup vs baseline: 10.2336x; 1.2036x over previous
import functools

import jax
import jax.numpy as jnp
from jax import lax
from jax.experimental import pallas as pl
from jax.experimental.pallas import tpu as pltpu
from jax.experimental.pallas import tpu_sc as plsc

N = 50000
E = 800000
D_IN = 128
H = 64
C = 4
B = 64

HALF = 25088
NP = 2 * HALF
NQ = 8
QR = NP // NQ
TRASH_Q = QR
ACC_R = 6400
BLK = 512
NBLK = NP // BLK

NTILE = 16
NCORE = 2
NW = NTILE * NCORE
EPAD = 800768
EPW = EPAD // NW
KCH = 128
R_CAP = 4096
PEND = 272
ZPT = ACC_R // NTILE
WPT = QR // NTILE

_mesh = plsc.VectorSubcoreMesh(core_axis_name="c", subcore_axis_name="s")


@functools.partial(
    pl.kernel,
    out_type=[
        jax.ShapeDtypeStruct((NQ * NW * R_CAP,), jnp.int32),
        jax.ShapeDtypeStruct((NQ * NW * R_CAP,), jnp.int32),
        jax.ShapeDtypeStruct((NW * 128,), jnp.int32),
    ],
    mesh=_mesh,
    scratch_types=[
        pltpu.VMEM((EPW,), jnp.int32),
        pltpu.VMEM((EPW,), jnp.int32),
    ] + [pltpu.VMEM((PEND,), jnp.int32)] * 16 + [
        pltpu.VMEM((KCH,), jnp.int32),
        pltpu.SMEM((16,), jnp.int32),
    ],
    compiler_params=pltpu.CompilerParams(needs_layout_passes=False),
)
def _part_kernel(src_hbm, dst_hbm, gsrc_hbm, ldst_hbm, counts_hbm,
                 sbuf, dbuf, *rest):
    pends = rest[:16]
    cbuf = rest[16]
    cnts = rest[17]
    psrcs = pends[:NQ]
    pdsts = pends[NQ:]
    wid = lax.axis_index("s") * NCORE + lax.axis_index("c")
    pltpu.sync_copy(src_hbm.at[pl.ds(pl.multiple_of(wid * EPW, 128), EPW)], sbuf)
    pltpu.sync_copy(dst_hbm.at[pl.ds(pl.multiple_of(wid * EPW, 128), EPW)], dbuf)
    for b in range(16):
        cnts[b] = 0

    def vec(i, _):
        sl = pl.ds(i * 16, 16)
        sv = sbuf[sl]
        dv = dbuf[sl]
        g = jnp.where(sv >= N // 2, sv + (HALF - N // 2), sv)
        gd = jnp.where(dv >= N // 2, dv + (HALF - N // 2), dv)
        q = (gd >= QR).astype(jnp.int32)
        for k in range(2, NQ):
            q = q + (gd >= k * QR).astype(jnp.int32)
        valid = dv >= 0
        for b in range(NQ):
            m = (q == b) & valid
            mi = m.astype(jnp.int32)
            psrc = psrcs[b]
            pdst = pdsts[b]
            cnt = cnts[b]
            rank = cnt + plsc.cumsum(mi) - 1
            plsc.store_scatter(psrc, [rank], g, mask=m)
            plsc.store_scatter(pdst, [rank], gd - b * QR, mask=m)
            cnt = cnt + jnp.sum(mi)
            cnts[b] = cnt

            @pl.when(cnt >= KCH)
            def _():
                out = pl.multiple_of(
                    jnp.minimum(cnts[NQ + b], R_CAP - KCH), KCH)
                rbase = pl.multiple_of((b * NW + wid) * R_CAP, KCH)
                pltpu.sync_copy(psrc.at[pl.ds(0, KCH)],
                                gsrc_hbm.at[pl.ds(rbase + out, KCH)])
                pltpu.sync_copy(pdst.at[pl.ds(0, KCH)],
                                ldst_hbm.at[pl.ds(rbase + out, KCH)])
                psrc[pl.ds(0, 16)] = psrc[pl.ds(KCH, 16)]
                pdst[pl.ds(0, 16)] = pdst[pl.ds(KCH, 16)]
                cnts[b] = cnt - KCH
                cnts[NQ + b] = out + KCH

        return 0

    lax.fori_loop(0, EPW // 16, vec, 0)

    zsrc = jnp.zeros((16,), jnp.int32)
    ztrash = jnp.full((16,), TRASH_Q, jnp.int32)
    lanes = lax.iota(jnp.int32, 16)
    cvec = jnp.zeros((16,), jnp.int32)
    for b in range(NQ):
        psrc = psrcs[b]
        pdst = pdsts[b]
        cnt = cnts[b]
        for i in range(8):
            fidx = cnt + i * 16 + lanes
            plsc.store_scatter(psrc, [fidx], zsrc)
            plsc.store_scatter(pdst, [fidx], ztrash)
        out = pl.multiple_of(jnp.minimum(cnts[NQ + b], R_CAP - KCH), KCH)
        rbase = pl.multiple_of((b * NW + wid) * R_CAP, KCH)
        pltpu.sync_copy(psrc.at[pl.ds(0, KCH)],
                        gsrc_hbm.at[pl.ds(rbase + out, KCH)])
        pltpu.sync_copy(pdst.at[pl.ds(0, KCH)],
                        ldst_hbm.at[pl.ds(rbase + out, KCH)])
        cvec = jnp.where(lanes == b, jnp.minimum(cnts[NQ + b], R_CAP - KCH) + KCH, cvec)
    for i in range(KCH // 16):
        cbuf[pl.ds(i * 16, 16)] = cvec
    pltpu.sync_copy(cbuf, counts_hbm.at[pl.ds(pl.multiple_of(wid * 128, 128), KCH)])


def _zero_fill(buf, nrow, val=0.0):
    def row(r, _):
        for jj in range(8):
            buf[r, pl.ds(jj * 16, 16)] = jnp.full((16,), val, jnp.float32)
        return 0

    lax.fori_loop(0, nrow, row, 0)


def _zero_acc(acc, zbuf, s):
    base = s * ZPT
    for k in range(ZPT // KCH):
        pltpu.sync_copy(zbuf, acc.at[pl.ds(base + k * KCH, KCH)])
    rem = ZPT % KCH
    if rem:
        pltpu.sync_copy(zbuf.at[pl.ds(0, rem)],
                        acc.at[pl.ds(base + (ZPT // KCH) * KCH, rem)])


@functools.partial(
    pl.kernel,
    out_type=jax.ShapeDtypeStruct((NP, 2 * H), jnp.float32),
    mesh=_mesh,
    scratch_types=[
        pltpu.VMEM_SHARED((ACC_R, 2 * H), jnp.float32),
        pltpu.VMEM((KCH, 2 * H), jnp.float32),
        pltpu.VMEM((KCH, 2 * H), jnp.float32),
        pltpu.VMEM((NW * 128,), jnp.int32),
        pltpu.VMEM((KCH,), jnp.int32),
    ],
    compiler_params=pltpu.CompilerParams(needs_layout_passes=False),
)
def _deg_kernel(ldst_hbm, counts_hbm, out_hbm, acc, obuf, zbuf, cvm, sidx):
    c = lax.axis_index("c")
    s = lax.axis_index("s")
    pltpu.sync_copy(counts_hbm, cvm)
    _zero_fill(obuf, KCH, 1.0)
    _zero_fill(zbuf, KCH, 0.0)

    for qq in range(NQ // 2):
        q = c * (NQ // 2) + qq
        _zero_acc(acc, zbuf, s)
        plsc.subcore_barrier()
        for rr in range(2):
            reg = s * 2 + rr
            crow = cvm[pl.ds(pl.multiple_of(reg * 128, 128), 16)]
            lanes = lax.iota(jnp.int32, 16)
            nch = jnp.sum(jnp.where(lanes == q, crow, 0)) // KCH
            nch = jnp.minimum(nch, R_CAP // KCH)
            rbase = pl.multiple_of((q * NW + reg) * R_CAP, KCH)

            def chunk(i, _):
                off = pl.multiple_of(rbase + i * KCH, KCH)
                pltpu.sync_copy(ldst_hbm.at[pl.ds(off, KCH)], sidx)
                pltpu.sync_copy(obuf, acc.at[sidx], add=True)
                return 0

            lax.fori_loop(0, nch, chunk, 0)
        plsc.subcore_barrier()
        pltpu.sync_copy(acc.at[pl.ds(s * WPT, WPT)],
                        out_hbm.at[pl.ds(q * QR + s * WPT, WPT)])
        plsc.subcore_barrier()


@functools.partial(
    pl.kernel,
    out_type=jax.ShapeDtypeStruct((NP, 2 * H), jnp.float32),
    mesh=_mesh,
    scratch_types=[
        pltpu.VMEM_SHARED((ACC_R, 2 * H), jnp.float32),
        pltpu.VMEM((KCH, 2 * H), jnp.float32),
        pltpu.VMEM((KCH, 2 * H), jnp.float32),
        pltpu.VMEM((KCH, 2 * H), jnp.float32),
        pltpu.VMEM((NW * 128,), jnp.int32),
        pltpu.VMEM((R_CAP,), jnp.int32),
        pltpu.VMEM((R_CAP,), jnp.int32),
        pltpu.VMEM((KCH,), jnp.int32),
        pltpu.VMEM((KCH,), jnp.int32),
        pltpu.SemaphoreType.DMA,
        pltpu.SemaphoreType.DMA,
    ],
    compiler_params=pltpu.CompilerParams(needs_layout_passes=False),
)
def _seg_kernel(gsrc_hbm, ldst_hbm, counts_hbm, xs_hbm, out_hbm,
                acc, rows0, rows1, zbuf, cvm, gbuf, sbuf, sx0, sx1,
                sem0, sem1):
    c = lax.axis_index("c")
    s = lax.axis_index("s")
    pltpu.sync_copy(counts_hbm, cvm)
    _zero_fill(zbuf, KCH, 0.0)

    for qq in range(NQ // 2):
        q = c * (NQ // 2) + qq
        _zero_acc(acc, zbuf, s)
        plsc.subcore_barrier()
        for rr in range(2):
            reg = s * 2 + rr
            crow = cvm[pl.ds(pl.multiple_of(reg * 128, 128), 16)]
            lanes = lax.iota(jnp.int32, 16)
            nch = jnp.sum(jnp.where(lanes == q, crow, 0)) // KCH
            nch = jnp.minimum(nch, R_CAP // KCH)
            rbase = pl.multiple_of((q * NW + reg) * R_CAP, KCH)
            nw_ = pl.multiple_of(nch * KCH, KCH)
            rowset = (rows0, rows1)
            sxset = (sx0, sx1)
            semset = (sem0, sem1)

            @pl.when(nch > 0)
            def _():
                pltpu.sync_copy(gsrc_hbm.at[pl.ds(rbase, R_CAP)], gbuf)
                pltpu.sync_copy(ldst_hbm.at[pl.ds(rbase, R_CAP)], sbuf)
                pltpu.async_copy(xs_hbm.at[gbuf.at[pl.ds(0, KCH)]],
                                 rows0, sem0)

                def chunk(i, _):
                    for par in range(2):
                        @pl.when((i % 2) == par)
                        def _():
                            rows = rowset[par]
                            sx = sxset[par]
                            sem = semset[par]

                            @pl.when(i + 1 < nch)
                            def _():
                                noff = pl.multiple_of((i + 1) * KCH, KCH)
                                pltpu.async_copy(
                                    xs_hbm.at[gbuf.at[pl.ds(noff, KCH)]],
                                    rowset[1 - par], semset[1 - par])

                            coff = pl.multiple_of(i * KCH, KCH)
                            for jj in range(KCH // 16):
                                sx[pl.ds(jj * 16, 16)] = (
                                    sbuf[pl.ds(coff + jj * 16, 16)])
                            pltpu.make_async_copy(
                                xs_hbm.at[gbuf.at[pl.ds(coff, KCH)]],
                                rows, sem).wait()
                            pltpu.sync_copy(rows, acc.at[sx], add=True)
                    return 0

                lax.fori_loop(0, nch, chunk, 0)
        plsc.subcore_barrier()
        pltpu.sync_copy(acc.at[pl.ds(s * WPT, WPT)],
                        out_hbm.at[pl.ds(q * QR + s * WPT, WPT)])
        plsc.subcore_barrier()


def _tcb_body(x_ref, hist_ref, w1_ref, xs_ref, dinv_ref):
    deg = hist_ref[:, :1] + 1.0
    dinv = lax.rsqrt(deg)
    xw = jnp.dot(x_ref[...], w1_ref[...], preferred_element_type=jnp.float32)
    xs_ref[...] = jnp.concatenate(
        [xw * dinv, jnp.zeros((BLK, H), jnp.float32)], axis=1)
    dinv_ref[...] = dinv


def _tcb(x_pad, hist, W1):
    return pl.pallas_call(
        _tcb_body,
        grid=(NBLK,),
        in_specs=[
            pl.BlockSpec((BLK, D_IN), lambda i: (i, 0)),
            pl.BlockSpec((BLK, 2 * H), lambda i: (i, 0)),
            pl.BlockSpec((D_IN, H), lambda i: (0, 0)),
        ],
        out_specs=[
            pl.BlockSpec((BLK, 2 * H), lambda i: (i, 0)),
            pl.BlockSpec((BLK, 1), lambda i: (i, 0)),
        ],
        out_shape=[
            jax.ShapeDtypeStruct((NP, 2 * H), jnp.float32),
            jax.ShapeDtypeStruct((NP, 1), jnp.float32),
        ],
    )(x_pad, hist, W1)


def _tcc_body(seg_ref, xs_ref, dinv_ref, b_ref, w2_ref, out_ref):
    dinv = dinv_ref[...]
    h = jax.nn.relu(dinv * (seg_ref[:, :H] + xs_ref[:, :H]) + b_ref[...])
    xw = jnp.dot(h, w2_ref[...], preferred_element_type=jnp.float32)
    out_ref[...] = jnp.concatenate(
        [xw * dinv, jnp.zeros((BLK, H), jnp.float32)], axis=1)


def _tcc(seg1, xs1, dinv, b1, W2):
    return pl.pallas_call(
        _tcc_body,
        grid=(NBLK,),
        in_specs=[
            pl.BlockSpec((BLK, 2 * H), lambda i: (i, 0)),
            pl.BlockSpec((BLK, 2 * H), lambda i: (i, 0)),
            pl.BlockSpec((BLK, 1), lambda i: (i, 0)),
            pl.BlockSpec((1, H), lambda i: (0, 0)),
            pl.BlockSpec((H, H), lambda i: (0, 0)),
        ],
        out_specs=pl.BlockSpec((BLK, 2 * H), lambda i: (i, 0)),
        out_shape=jax.ShapeDtypeStruct((NP, 2 * H), jnp.float32),
    )(seg1, xs1, dinv, b1, W2)


def _tcd_body(seg_ref, xs_ref, dinv_ref, b_ref, batch_ref,
              wf1_ref, bf1_ref, wf2_ref, bf2_ref, out_ref, sums, cnts):
    i = pl.program_id(0)

    @pl.when(i == 0)
    def _():
        sums[...] = jnp.zeros_like(sums)
        cnts[...] = jnp.zeros_like(cnts)

    h = jax.nn.relu(dinv_ref[...] * (seg_ref[:, :H] + xs_ref[:, :H])
                    + b_ref[...])
    bv = batch_ref[0, 0, :]
    m = (bv[:, None] == lax.broadcasted_iota(jnp.int32, (BLK, B), 1))
    mf = m.astype(jnp.float32)
    dn = (((0,), (0,)), ((), ()))
    sums[...] += lax.dot_general(mf, h, dn, preferred_element_type=jnp.float32)
    cnts[...] += lax.dot_general(mf, jnp.ones((BLK, 8), jnp.float32), dn,
                                 preferred_element_type=jnp.float32)

    @pl.when(i == NBLK - 1)
    def _():
        g = sums[...] / jnp.maximum(cnts[...][:, :1], 1.0)
        a = jax.nn.relu(
            jnp.dot(g, wf1_ref[...], preferred_element_type=jnp.float32)
            + bf1_ref[...])
        out_ref[...] = (
            jnp.dot(a, wf2_ref[...], preferred_element_type=jnp.float32)
            + bf2_ref[...])


def _tcd(seg2, xs2, dinv, b2, batch3, Wf1, bf1, Wf2, bf2):
    return pl.pallas_call(
        _tcd_body,
        grid=(NBLK,),
        in_specs=[
            pl.BlockSpec((BLK, 2 * H), lambda i: (i, 0)),
            pl.BlockSpec((BLK, 2 * H), lambda i: (i, 0)),
            pl.BlockSpec((BLK, 1), lambda i: (i, 0)),
            pl.BlockSpec((1, H), lambda i: (0, 0)),
            pl.BlockSpec((1, 1, BLK), lambda i: (i, 0, 0)),
            pl.BlockSpec((H, H), lambda i: (0, 0)),
            pl.BlockSpec((1, H), lambda i: (0, 0)),
            pl.BlockSpec((H, C), lambda i: (0, 0)),
            pl.BlockSpec((1, C), lambda i: (0, 0)),
        ],
        out_specs=pl.BlockSpec((B, C), lambda i: (0, 0)),
        out_shape=jax.ShapeDtypeStruct((B, C), jnp.float32),
        scratch_shapes=[
            pltpu.VMEM((B, B), jnp.float32),
            pltpu.VMEM((B, 8), jnp.float32),
        ],
    )(seg2, xs2, dinv, b2, batch3, Wf1, bf1, Wf2, bf2)


@jax.jit
def kernel(x, edge_index, batch, W1, b1, W2, b2, Wf1, bf1, Wf2, bf2):
    src = edge_index[0].astype(jnp.int32)
    dst = edge_index[1].astype(jnp.int32)
    pad_e = EPAD - E
    src_pad = jnp.concatenate([src, jnp.zeros((pad_e,), jnp.int32)])
    dst_pad = jnp.concatenate([dst, jnp.full((pad_e,), -1, jnp.int32)])

    half = N // 2
    gap = HALF - half
    x_pad = jnp.concatenate([
        x[:half], jnp.zeros((gap, D_IN), jnp.float32),
        x[half:], jnp.zeros((gap, D_IN), jnp.float32)])
    b32 = batch.astype(jnp.int32)
    batch_pad = jnp.concatenate([
        b32[:half], jnp.full((gap,), B, jnp.int32),
        b32[half:], jnp.full((gap,), B, jnp.int32)])
    batch3 = batch_pad.reshape(NBLK, 1, BLK)

    b1r = b1.reshape(1, H)
    b2r = b2.reshape(1, H)
    bf1r = bf1.reshape(1, H)
    bf2r = bf2.reshape(1, C)

    gsrc, ldst, counts = _part_kernel(src_pad, dst_pad)
    hist = _deg_kernel(ldst, counts)
    xs1, dinv = _tcb(x_pad, hist, W1)
    seg1 = _seg_kernel(gsrc, ldst, counts, xs1)
    xs2 = _tcc(seg1, xs1, dinv, b1r, W2)
    seg2 = _seg_kernel(gsrc, ldst, counts, xs2)
    return _tcd(seg2, xs2, dinv, b2r, batch3, Wf1, bf1r, Wf2, bf2r)


# speedup vs baseline: 10.5588x; 1.0318x over previous
import functools

import jax
import jax.numpy as jnp
from jax import lax
from jax.experimental import pallas as pl
from jax.experimental.pallas import tpu as pltpu
from jax.experimental.pallas import tpu_sc as plsc

N = 50000
E = 800000
D_IN = 128
H = 64
C = 4
B = 64

HALF = 25088
NP = 2 * HALF
NQ = 8
QR = NP // NQ
TRASH_Q = QR
ACC_R = 6400
BLK = 512
NBLK = NP // BLK

NTILE = 16
NCORE = 2
NW = NTILE * NCORE
EPAD = 800768
EPW = EPAD // NW
KCH = 128
R_CAP = 4096
PEND = 272
ZPT = ACC_R // NTILE
WPT = QR // NTILE

_mesh = plsc.VectorSubcoreMesh(core_axis_name="c", subcore_axis_name="s")


@functools.partial(
    pl.kernel,
    out_type=[
        jax.ShapeDtypeStruct((NQ * NW * R_CAP,), jnp.int32),
        jax.ShapeDtypeStruct((NQ * NW * R_CAP,), jnp.int32),
        jax.ShapeDtypeStruct((NW * 128,), jnp.int32),
    ],
    mesh=_mesh,
    scratch_types=[
        pltpu.VMEM((EPW,), jnp.int32),
        pltpu.VMEM((EPW,), jnp.int32),
    ] + [pltpu.VMEM((PEND,), jnp.int32)] * 16 + [
        pltpu.VMEM((KCH,), jnp.int32),
        pltpu.SMEM((16,), jnp.int32),
    ],
    compiler_params=pltpu.CompilerParams(needs_layout_passes=False),
)
def _part_kernel(src_hbm, dst_hbm, gsrc_hbm, ldst_hbm, counts_hbm,
                 sbuf, dbuf, *rest):
    pends = rest[:16]
    cbuf = rest[16]
    cnts = rest[17]
    psrcs = pends[:NQ]
    pdsts = pends[NQ:]
    wid = lax.axis_index("s") * NCORE + lax.axis_index("c")
    pltpu.sync_copy(src_hbm.at[pl.ds(pl.multiple_of(wid * EPW, 128), EPW)], sbuf)
    pltpu.sync_copy(dst_hbm.at[pl.ds(pl.multiple_of(wid * EPW, 128), EPW)], dbuf)
    for b in range(16):
        cnts[b] = 0

    def vec(i, _):
        sl = pl.ds(i * 16, 16)
        sv = sbuf[sl]
        dv = dbuf[sl]
        g = jnp.where(sv >= N // 2, sv + (HALF - N // 2), sv)
        gd = jnp.where(dv >= N // 2, dv + (HALF - N // 2), dv)
        q = (gd >= QR).astype(jnp.int32)
        for k in range(2, NQ):
            q = q + (gd >= k * QR).astype(jnp.int32)
        valid = dv >= 0
        for b in range(NQ):
            m = (q == b) & valid
            mi = m.astype(jnp.int32)
            psrc = psrcs[b]
            pdst = pdsts[b]
            cnt = cnts[b]
            rank = cnt + plsc.cumsum(mi) - 1
            plsc.store_scatter(psrc, [rank], g, mask=m)
            plsc.store_scatter(pdst, [rank], gd - b * QR, mask=m)
            cnt = cnt + jnp.sum(mi)
            cnts[b] = cnt

            @pl.when(cnt >= KCH)
            def _():
                out = pl.multiple_of(
                    jnp.minimum(cnts[NQ + b], R_CAP - KCH), KCH)
                rbase = pl.multiple_of((b * NW + wid) * R_CAP, KCH)
                pltpu.sync_copy(psrc.at[pl.ds(0, KCH)],
                                gsrc_hbm.at[pl.ds(rbase + out, KCH)])
                pltpu.sync_copy(pdst.at[pl.ds(0, KCH)],
                                ldst_hbm.at[pl.ds(rbase + out, KCH)])
                psrc[pl.ds(0, 16)] = psrc[pl.ds(KCH, 16)]
                pdst[pl.ds(0, 16)] = pdst[pl.ds(KCH, 16)]
                cnts[b] = cnt - KCH
                cnts[NQ + b] = out + KCH

        return 0

    lax.fori_loop(0, EPW // 16, vec, 0)

    zsrc = jnp.zeros((16,), jnp.int32)
    ztrash = jnp.full((16,), TRASH_Q, jnp.int32)
    lanes = lax.iota(jnp.int32, 16)
    cvec = jnp.zeros((16,), jnp.int32)
    for b in range(NQ):
        psrc = psrcs[b]
        pdst = pdsts[b]
        cnt = cnts[b]
        for i in range(8):
            fidx = cnt + i * 16 + lanes
            plsc.store_scatter(psrc, [fidx], zsrc)
            plsc.store_scatter(pdst, [fidx], ztrash)
        out = pl.multiple_of(jnp.minimum(cnts[NQ + b], R_CAP - KCH), KCH)
        rbase = pl.multiple_of((b * NW + wid) * R_CAP, KCH)
        pltpu.sync_copy(psrc.at[pl.ds(0, KCH)],
                        gsrc_hbm.at[pl.ds(rbase + out, KCH)])
        pltpu.sync_copy(pdst.at[pl.ds(0, KCH)],
                        ldst_hbm.at[pl.ds(rbase + out, KCH)])
        cvec = jnp.where(lanes == b, jnp.minimum(cnts[NQ + b], R_CAP - KCH) + KCH, cvec)
    for i in range(KCH // 16):
        cbuf[pl.ds(i * 16, 16)] = cvec
    pltpu.sync_copy(cbuf, counts_hbm.at[pl.ds(pl.multiple_of(wid * 128, 128), KCH)])


def _zero_fill(buf, nrow, val=0.0):
    def row(r, _):
        for jj in range(8):
            buf[r, pl.ds(jj * 16, 16)] = jnp.full((16,), val, jnp.float32)
        return 0

    lax.fori_loop(0, nrow, row, 0)


def _zero_acc(acc, zbuf, s):
    base = s * ZPT
    for k in range(ZPT // KCH):
        pltpu.sync_copy(zbuf, acc.at[pl.ds(base + k * KCH, KCH)])
    rem = ZPT % KCH
    if rem:
        pltpu.sync_copy(zbuf.at[pl.ds(0, rem)],
                        acc.at[pl.ds(base + (ZPT // KCH) * KCH, rem)])


@functools.partial(
    pl.kernel,
    out_type=jax.ShapeDtypeStruct((NP, 2 * H), jnp.float32),
    mesh=_mesh,
    scratch_types=[
        pltpu.VMEM_SHARED((ACC_R, 2 * H), jnp.float32),
        pltpu.VMEM((KCH, 2 * H), jnp.float32),
        pltpu.VMEM((KCH, 2 * H), jnp.float32),
        pltpu.VMEM((NW * 128,), jnp.int32),
        pltpu.VMEM((R_CAP,), jnp.int32),
        pltpu.VMEM((KCH,), jnp.int32),
    ],
    compiler_params=pltpu.CompilerParams(needs_layout_passes=False),
)
def _deg_kernel(ldst_hbm, counts_hbm, out_hbm, acc, obuf, zbuf, cvm, sbuf,
                sidx):
    c = lax.axis_index("c")
    s = lax.axis_index("s")
    pltpu.sync_copy(counts_hbm, cvm)
    _zero_fill(obuf, KCH, 1.0)
    _zero_fill(zbuf, KCH, 0.0)

    for qq in range(NQ // 2):
        q = c * (NQ // 2) + qq
        _zero_acc(acc, zbuf, s)
        plsc.subcore_barrier()
        for rr in range(2):
            reg = s * 2 + rr
            crow = cvm[pl.ds(pl.multiple_of(reg * 128, 128), 16)]
            lanes = lax.iota(jnp.int32, 16)
            nch = jnp.sum(jnp.where(lanes == q, crow, 0)) // KCH
            nch = jnp.minimum(nch, R_CAP // KCH)
            rbase = pl.multiple_of((q * NW + reg) * R_CAP, KCH)

            @pl.when(nch > 0)
            def _():
                pltpu.sync_copy(ldst_hbm.at[pl.ds(rbase, R_CAP)], sbuf)

                def chunk(i, _):
                    coff = pl.multiple_of(i * KCH, KCH)
                    for jj in range(KCH // 16):
                        sidx[pl.ds(jj * 16, 16)] = (
                            sbuf[pl.ds(coff + jj * 16, 16)])
                    pltpu.sync_copy(obuf, acc.at[sidx], add=True)
                    return 0

                lax.fori_loop(0, nch, chunk, 0)
        plsc.subcore_barrier()
        pltpu.sync_copy(acc.at[pl.ds(s * WPT, WPT)],
                        out_hbm.at[pl.ds(q * QR + s * WPT, WPT)])
        plsc.subcore_barrier()


@functools.partial(
    pl.kernel,
    out_type=jax.ShapeDtypeStruct((NP, 2 * H), jnp.float32),
    mesh=_mesh,
    scratch_types=[
        pltpu.VMEM_SHARED((ACC_R, 2 * H), jnp.float32),
        pltpu.VMEM((KCH, 2 * H), jnp.float32),
        pltpu.VMEM((KCH, 2 * H), jnp.float32),
        pltpu.VMEM((KCH, 2 * H), jnp.float32),
        pltpu.VMEM((NW * 128,), jnp.int32),
        pltpu.VMEM((R_CAP,), jnp.int32),
        pltpu.VMEM((R_CAP,), jnp.int32),
        pltpu.VMEM((KCH,), jnp.int32),
        pltpu.VMEM((KCH,), jnp.int32),
        pltpu.SemaphoreType.DMA,
        pltpu.SemaphoreType.DMA,
    ],
    compiler_params=pltpu.CompilerParams(needs_layout_passes=False),
)
def _seg_kernel(gsrc_hbm, ldst_hbm, counts_hbm, xs_hbm, out_hbm,
                acc, rows0, rows1, zbuf, cvm, gbuf, sbuf, sx0, sx1,
                sem0, sem1):
    c = lax.axis_index("c")
    s = lax.axis_index("s")
    pltpu.sync_copy(counts_hbm, cvm)
    _zero_fill(zbuf, KCH, 0.0)

    for qq in range(NQ // 2):
        q = c * (NQ // 2) + qq
        _zero_acc(acc, zbuf, s)
        plsc.subcore_barrier()
        for rr in range(2):
            reg = s * 2 + rr
            crow = cvm[pl.ds(pl.multiple_of(reg * 128, 128), 16)]
            lanes = lax.iota(jnp.int32, 16)
            nch = jnp.sum(jnp.where(lanes == q, crow, 0)) // KCH
            nch = jnp.minimum(nch, R_CAP // KCH)
            rbase = pl.multiple_of((q * NW + reg) * R_CAP, KCH)
            nw_ = pl.multiple_of(nch * KCH, KCH)
            rowset = (rows0, rows1)
            sxset = (sx0, sx1)
            semset = (sem0, sem1)

            @pl.when(nch > 0)
            def _():
                pltpu.sync_copy(gsrc_hbm.at[pl.ds(rbase, R_CAP)], gbuf)
                pltpu.sync_copy(ldst_hbm.at[pl.ds(rbase, R_CAP)], sbuf)
                pltpu.async_copy(xs_hbm.at[gbuf.at[pl.ds(0, KCH)]],
                                 rows0, sem0)

                def chunk(i, _):
                    for par in range(2):
                        @pl.when((i % 2) == par)
                        def _():
                            rows = rowset[par]
                            sx = sxset[par]
                            sem = semset[par]

                            @pl.when(i + 1 < nch)
                            def _():
                                noff = pl.multiple_of((i + 1) * KCH, KCH)
                                pltpu.async_copy(
                                    xs_hbm.at[gbuf.at[pl.ds(noff, KCH)]],
                                    rowset[1 - par], semset[1 - par])

                            coff = pl.multiple_of(i * KCH, KCH)
                            for jj in range(KCH // 16):
                                sx[pl.ds(jj * 16, 16)] = (
                                    sbuf[pl.ds(coff + jj * 16, 16)])
                            pltpu.make_async_copy(
                                xs_hbm.at[gbuf.at[pl.ds(coff, KCH)]],
                                rows, sem).wait()
                            pltpu.sync_copy(rows, acc.at[sx], add=True)
                    return 0

                lax.fori_loop(0, nch, chunk, 0)
        plsc.subcore_barrier()
        pltpu.sync_copy(acc.at[pl.ds(s * WPT, WPT)],
                        out_hbm.at[pl.ds(q * QR + s * WPT, WPT)])
        plsc.subcore_barrier()


def _tcb_body(x_ref, hist_ref, w1_ref, xs_ref, dinv_ref):
    deg = hist_ref[:, :1] + 1.0
    dinv = lax.rsqrt(deg)
    xw = jnp.dot(x_ref[...], w1_ref[...], preferred_element_type=jnp.float32)
    xs_ref[...] = jnp.concatenate(
        [xw * dinv, jnp.zeros((BLK, H), jnp.float32)], axis=1)
    dinv_ref[...] = dinv


def _tcb(x_pad, hist, W1):
    return pl.pallas_call(
        _tcb_body,
        grid=(NBLK,),
        in_specs=[
            pl.BlockSpec((BLK, D_IN), lambda i: (i, 0)),
            pl.BlockSpec((BLK, 2 * H), lambda i: (i, 0)),
            pl.BlockSpec((D_IN, H), lambda i: (0, 0)),
        ],
        out_specs=[
            pl.BlockSpec((BLK, 2 * H), lambda i: (i, 0)),
            pl.BlockSpec((BLK, 1), lambda i: (i, 0)),
        ],
        out_shape=[
            jax.ShapeDtypeStruct((NP, 2 * H), jnp.float32),
            jax.ShapeDtypeStruct((NP, 1), jnp.float32),
        ],
    )(x_pad, hist, W1)


def _tcc_body(seg_ref, xs_ref, dinv_ref, b_ref, w2_ref, out_ref):
    dinv = dinv_ref[...]
    h = jax.nn.relu(dinv * (seg_ref[:, :H] + xs_ref[:, :H]) + b_ref[...])
    xw = jnp.dot(h, w2_ref[...], preferred_element_type=jnp.float32)
    out_ref[...] = jnp.concatenate(
        [xw * dinv, jnp.zeros((BLK, H), jnp.float32)], axis=1)


def _tcc(seg1, xs1, dinv, b1, W2):
    return pl.pallas_call(
        _tcc_body,
        grid=(NBLK,),
        in_specs=[
            pl.BlockSpec((BLK, 2 * H), lambda i: (i, 0)),
            pl.BlockSpec((BLK, 2 * H), lambda i: (i, 0)),
            pl.BlockSpec((BLK, 1), lambda i: (i, 0)),
            pl.BlockSpec((1, H), lambda i: (0, 0)),
            pl.BlockSpec((H, H), lambda i: (0, 0)),
        ],
        out_specs=pl.BlockSpec((BLK, 2 * H), lambda i: (i, 0)),
        out_shape=jax.ShapeDtypeStruct((NP, 2 * H), jnp.float32),
    )(seg1, xs1, dinv, b1, W2)


def _tcd_body(seg_ref, xs_ref, dinv_ref, b_ref, batch_ref,
              wf1_ref, bf1_ref, wf2_ref, bf2_ref, out_ref, sums, cnts):
    i = pl.program_id(0)

    @pl.when(i == 0)
    def _():
        sums[...] = jnp.zeros_like(sums)
        cnts[...] = jnp.zeros_like(cnts)

    h = jax.nn.relu(dinv_ref[...] * (seg_ref[:, :H] + xs_ref[:, :H])
                    + b_ref[...])
    bv = batch_ref[0, 0, :]
    m = (bv[:, None] == lax.broadcasted_iota(jnp.int32, (BLK, B), 1))
    mf = m.astype(jnp.float32)
    dn = (((0,), (0,)), ((), ()))
    sums[...] += lax.dot_general(mf, h, dn, preferred_element_type=jnp.float32)
    cnts[...] += lax.dot_general(mf, jnp.ones((BLK, 8), jnp.float32), dn,
                                 preferred_element_type=jnp.float32)

    @pl.when(i == NBLK - 1)
    def _():
        g = sums[...] / jnp.maximum(cnts[...][:, :1], 1.0)
        a = jax.nn.relu(
            jnp.dot(g, wf1_ref[...], preferred_element_type=jnp.float32)
            + bf1_ref[...])
        out_ref[...] = (
            jnp.dot(a, wf2_ref[...], preferred_element_type=jnp.float32)
            + bf2_ref[...])


def _tcd(seg2, xs2, dinv, b2, batch3, Wf1, bf1, Wf2, bf2):
    return pl.pallas_call(
        _tcd_body,
        grid=(NBLK,),
        in_specs=[
            pl.BlockSpec((BLK, 2 * H), lambda i: (i, 0)),
            pl.BlockSpec((BLK, 2 * H), lambda i: (i, 0)),
            pl.BlockSpec((BLK, 1), lambda i: (i, 0)),
            pl.BlockSpec((1, H), lambda i: (0, 0)),
            pl.BlockSpec((1, 1, BLK), lambda i: (i, 0, 0)),
            pl.BlockSpec((H, H), lambda i: (0, 0)),
            pl.BlockSpec((1, H), lambda i: (0, 0)),
            pl.BlockSpec((H, C), lambda i: (0, 0)),
            pl.BlockSpec((1, C), lambda i: (0, 0)),
        ],
        out_specs=pl.BlockSpec((B, C), lambda i: (0, 0)),
        out_shape=jax.ShapeDtypeStruct((B, C), jnp.float32),
        scratch_shapes=[
            pltpu.VMEM((B, B), jnp.float32),
            pltpu.VMEM((B, 8), jnp.float32),
        ],
    )(seg2, xs2, dinv, b2, batch3, Wf1, bf1, Wf2, bf2)


@jax.jit
def kernel(x, edge_index, batch, W1, b1, W2, b2, Wf1, bf1, Wf2, bf2):
    src = edge_index[0].astype(jnp.int32)
    dst = edge_index[1].astype(jnp.int32)
    pad_e = EPAD - E
    src_pad = jnp.concatenate([src, jnp.zeros((pad_e,), jnp.int32)])
    dst_pad = jnp.concatenate([dst, jnp.full((pad_e,), -1, jnp.int32)])

    half = N // 2
    gap = HALF - half
    x_pad = jnp.concatenate([
        x[:half], jnp.zeros((gap, D_IN), jnp.float32),
        x[half:], jnp.zeros((gap, D_IN), jnp.float32)])
    b32 = batch.astype(jnp.int32)
    batch_pad = jnp.concatenate([
        b32[:half], jnp.full((gap,), B, jnp.int32),
        b32[half:], jnp.full((gap,), B, jnp.int32)])
    batch3 = batch_pad.reshape(NBLK, 1, BLK)

    b1r = b1.reshape(1, H)
    b2r = b2.reshape(1, H)
    bf1r = bf1.reshape(1, H)
    bf2r = bf2.reshape(1, C)

    gsrc, ldst, counts = _part_kernel(src_pad, dst_pad)
    hist = _deg_kernel(ldst, counts)
    xs1, dinv = _tcb(x_pad, hist, W1)
    seg1 = _seg_kernel(gsrc, ldst, counts, xs1)
    xs2 = _tcc(seg1, xs1, dinv, b1r, W2)
    seg2 = _seg_kernel(gsrc, ldst, counts, xs2)
    return _tcd(seg2, xs2, dinv, b2r, batch3, Wf1, bf1r, Wf2, bf2r)
